# Initial kernel scaffold; baseline (speedup 1.0000x reference)
import jax
import jax.numpy as jnp
from jax.experimental import pallas as pl


def kernel(xin, codebooks, W_in, W_out):
    raise NotImplementedError("write your pallas kernel here")


# speedup vs baseline: 2.8731x; 2.8731x over previous
import jax
import jax.numpy as jnp
from jax.experimental import pallas as pl
from jax.experimental.pallas import tpu as pltpu

_SCALES = (1.0, 2.0, 4.0, 8.0)
_N_CODES = 1024
_CODE_DIM = 64
_HIDDEN = 512
_T_LEN = 2048
_BATCH = 16
_N_Q = 4
_TBLK = 512


def _rvq_kernel(x_ref, cb_ref, win_ref, wout_ref, zq_ref, codes_ref, loss_ref):
    f32 = jnp.float32
    r = x_ref[0]
    tblk = r.shape[1]
    zq_acc = jnp.zeros_like(r)
    loss_vec = jnp.zeros((1, tblk), dtype=f32)
    iota = jax.lax.broadcasted_iota(jnp.int32, (_N_CODES, tblk), 0)
    code_rows = []
    for i, s in enumerate(_SCALES):
        wi = win_ref[i]
        wo = wout_ref[i]
        cb = cb_ref[i]
        z_e = jax.lax.dot_general(
            wi, r, (((1,), (0,)), ((), ())), preferred_element_type=f32
        ) * (1.0 / s)
        scores = jax.lax.dot_general(
            cb, z_e, (((1,), (0,)), ((), ())), preferred_element_type=f32
        )
        cb_sq = jnp.sum(cb * cb, axis=1, keepdims=True)
        ze_sq = jnp.sum(z_e * z_e, axis=0, keepdims=True)
        dist = ze_sq - 2.0 * scores + cb_sq
        dmin = jnp.min(dist, axis=0, keepdims=True)
        idx = jnp.min(
            jnp.where(dist == dmin, iota, _N_CODES), axis=0, keepdims=True
        )
        onehot = (iota == idx).astype(f32)
        zq = jax.lax.dot_general(
            cb, onehot, (((0,), (0,)), ((), ())), preferred_element_type=f32
        )
        d = z_e - zq
        loss_vec = loss_vec + jnp.sum(d * d, axis=0, keepdims=True)
        zq_st = z_e + (zq - z_e)
        zq_i = jax.lax.dot_general(
            wo, zq_st, (((1,), (0,)), ((), ())), preferred_element_type=f32
        ) * s
        r = r - zq_i
        zq_acc = zq_acc + zq_i
        code_rows.append(idx)
    zq_ref[0] = zq_acc
    codes_ref[0] = jnp.concatenate(code_rows, axis=0)
    loss_ref[0] = jnp.full((8, 128), jnp.sum(loss_vec), dtype=f32)


def kernel(xin, codebooks, W_in, W_out):
    b, d, t = xin.shape
    n_tb = t // _TBLK
    grid = (b, n_tb)
    out_shape = [
        jax.ShapeDtypeStruct((b, d, t), jnp.float32),
        jax.ShapeDtypeStruct((b, _N_Q, t), jnp.int32),
        jax.ShapeDtypeStruct((b, 8 * n_tb, 128), jnp.float32),
    ]
    in_specs = [
        pl.BlockSpec((1, d, _TBLK), lambda i, j: (i, 0, j)),
        pl.BlockSpec((_N_Q, _N_CODES, _CODE_DIM), lambda i, j: (0, 0, 0)),
        pl.BlockSpec((_N_Q, _CODE_DIM, d), lambda i, j: (0, 0, 0)),
        pl.BlockSpec((_N_Q, d, _CODE_DIM), lambda i, j: (0, 0, 0)),
    ]
    out_specs = [
        pl.BlockSpec((1, d, _TBLK), lambda i, j: (i, 0, j)),
        pl.BlockSpec((1, _N_Q, _TBLK), lambda i, j: (i, 0, j)),
        pl.BlockSpec((1, 8, 128), lambda i, j: (i, j, 0)),
    ]
    z_q, codes_bqt, loss_parts = pl.pallas_call(
        _rvq_kernel,
        grid=grid,
        in_specs=in_specs,
        out_specs=out_specs,
        out_shape=out_shape,
        compiler_params=pltpu.CompilerParams(
            dimension_semantics=("parallel", "parallel"),
        ),
    )(xin, codebooks, W_in, W_out)
    codes = codes_bqt.transpose(1, 0, 2)
    total_sq = jnp.sum(loss_parts) / 1024.0
    loss = total_sq * (1.25 / (b * t * _CODE_DIM))
    return z_q, loss, codes
